# baseline (speedup 1.0000x reference)
import functools

import jax
import jax.numpy as jnp
import numpy as np
from jax.experimental import pallas as pl
from jax.experimental.pallas import tpu as pltpu

_VMEM = 56 * 1024 * 1024


def _cp(*sem):
    return pltpu.CompilerParams(dimension_semantics=sem,
                                vmem_limit_bytes=_VMEM)


def _s1_gram_kernel(p_ref, g_ref):
    @pl.when(pl.program_id(1) == 0)
    def _():
        g_ref[...] = jnp.zeros_like(g_ref)

    a = p_ref[...]
    G = jax.lax.dot_general(a, a, (((1,), (1,)), ((), ())),
                            preferred_element_type=jnp.float32)
    rs = jnp.sum(a.astype(jnp.float32), axis=1, keepdims=True)
    g_ref[0:27, 0:28] += jnp.concatenate([G, rs], axis=1)


def _s1_gram(pT, tm):
    K, M = pT.shape
    half = M // (2 * tm)
    out = pl.pallas_call(
        _s1_gram_kernel,
        out_shape=jax.ShapeDtypeStruct((64, 128), jnp.float32),
        grid=(2, half),
        in_specs=[pl.BlockSpec((K, tm), lambda c, i: (0, c * (M // (2 * tm)) + i))],
        out_specs=pl.BlockSpec((32, 128), lambda c, i: (c, 0)),
        compiler_params=_cp("parallel", "arbitrary"),
    )(pT)
    s = out[:32] + out[32:]
    return s[0:27, 0:27], s[0:27, 27]


def _s1_post_kernel(p_ref, w_ref, b_ref, sc_ref, sh_ref, o_ref, *, slope, dpad):
    g = pl.program_id(0) * pl.num_programs(1) + pl.program_id(1)
    dp = g % dpad
    interior = (dp >= 1) & (dp <= dpad - 2)

    @pl.when(interior)
    def _():
        z = jax.lax.dot_general(p_ref[...], w_ref[...], (((0,), (0,)), ((), ())),
                                preferred_element_type=jnp.float32)
        z = z + b_ref[...]
        y = z * sc_ref[...] + sh_ref[...]
        y = jnp.where(y >= 0, y, slope * y).astype(jnp.bfloat16)
        C = y.shape[1]
        y = y.reshape(128, 128, C)
        yr = y.reshape(64, 2, 64, 2, C)
        planes = []
        for ph in range(2):
            for pw in range(2):
                q = yr[:, 1 - ph, :, 1 - pw, :]
                zr = jnp.zeros((1, 64, C), jnp.bfloat16)
                q = jnp.concatenate([zr, q] if ph == 0 else [q, zr], axis=0)
                zc = jnp.zeros((65, 1, C), jnp.bfloat16)
                q = jnp.concatenate([zc, q] if pw == 0 else [q, zc], axis=1)
                planes.append(q)
        o_ref[0, :, 0] = jnp.stack(planes, axis=0)

    @pl.when(jnp.logical_not(interior))
    def _():
        o_ref[...] = jnp.zeros_like(o_ref)


def _s1_post(pT, wm, b, sc, sh, slope, N, D):
    dpad = D + 2
    G = N * dpad
    half = G // 2
    mblk = 128 * 128

    def pmap(c, i):
        g = c * half + i
        n, dp = g // dpad, g % dpad
        return (0, jnp.clip(n * D + dp - 1, 0, N * D - 1))

    def omap(c, i):
        g = c * half + i
        return (g // dpad, 0, g % dpad, 0, 0, 0)

    return pl.pallas_call(
        functools.partial(_s1_post_kernel, slope=slope, dpad=dpad),
        out_shape=jax.ShapeDtypeStruct((N, 4, dpad, 65, 65, 32), jnp.bfloat16),
        grid=(2, half),
        in_specs=[pl.BlockSpec((27, mblk), pmap),
                  pl.BlockSpec((27, 32), lambda c, i: (0, 0)),
                  pl.BlockSpec((1, 32), lambda c, i: (0, 0)),
                  pl.BlockSpec((1, 32), lambda c, i: (0, 0)),
                  pl.BlockSpec((1, 32), lambda c, i: (0, 0))],
        out_specs=pl.BlockSpec((1, 4, 1, 65, 65, 32), omap),
        compiler_params=_cp("parallel", "arbitrary"),
    )(pT, wm, b, sc, sh)


def _conv_s2_kernel(x0_ref, x1_ref, x2_ref, w_ref, b_ref, z_ref, s_ref, sq_ref,
                    patch_ref, acc_ref, *, Ho, Wo, Cin, Cout):
    @pl.when(pl.program_id(1) == 0)
    def _():
        s_ref[...] = jnp.zeros_like(s_ref)
        sq_ref[...] = jnp.zeros_like(sq_ref)

    acc_ref[...] = jnp.broadcast_to(b_ref[...], acc_ref.shape)
    M = Ho * Wo
    for kd, x_ref in enumerate((x0_ref, x1_ref, x2_ref)):
        for kh in range(3):
            for kw in range(3):
                p, h0, w0 = 2 * (kh % 2) + (kw % 2), kh // 2, kw // 2
                tap = x_ref[0, p, 0, h0:h0 + Ho, w0:w0 + Wo, :]
                c0 = (kh * 3 + kw) * Cin
                patch_ref[:, c0:c0 + Cin] = tap.reshape(M, Cin)
        acc_ref[...] += jnp.dot(patch_ref[...], w_ref[kd],
                                preferred_element_type=jnp.float32)

    acc = acc_ref[...]
    z_ref[0, 0] = acc.reshape(Ho, Wo, Cout).astype(jnp.bfloat16)
    zg = acc.reshape(-1, 8, Cout)
    s_ref[...] += jnp.sum(zg, axis=0)
    sq_ref[...] += jnp.sum(zg * zg, axis=0)


def _conv_s2(xp, wm, b, *, Do, Ho, Wo, Cin, Cout):
    N = xp.shape[0]
    _, _, _, Hp, Wp, _ = xp.shape
    half = (N * Do) // 2

    def xmapj(j):
        def m(c, i):
            g = c * half + i
            return (g // Do, 0, 2 * (g % Do) + j, 0, 0, 0)
        return m

    def zmap(c, i):
        g = c * half + i
        return (g // Do, g % Do, 0, 0, 0)

    kern = functools.partial(_conv_s2_kernel, Ho=Ho, Wo=Wo, Cin=Cin, Cout=Cout)
    return pl.pallas_call(
        kern,
        out_shape=(jax.ShapeDtypeStruct((N, Do, Ho, Wo, Cout), jnp.bfloat16),
                   jax.ShapeDtypeStruct((16, Cout), jnp.float32),
                   jax.ShapeDtypeStruct((16, Cout), jnp.float32)),
        grid=(2, half),
        in_specs=[pl.BlockSpec((1, 4, 1, Hp, Wp, Cin), xmapj(0)),
                  pl.BlockSpec((1, 4, 1, Hp, Wp, Cin), xmapj(1)),
                  pl.BlockSpec((1, 4, 1, Hp, Wp, Cin), xmapj(2)),
                  pl.BlockSpec((3, 9 * Cin, Cout), lambda c, i: (0, 0, 0)),
                  pl.BlockSpec((1, Cout), lambda c, i: (0, 0))],
        out_specs=(pl.BlockSpec((1, 1, Ho, Wo, Cout), zmap),
                   pl.BlockSpec((8, Cout), lambda c, i: (c, 0)),
                   pl.BlockSpec((8, Cout), lambda c, i: (c, 0))),
        scratch_shapes=[pltpu.VMEM((Ho * Wo, 9 * Cin), jnp.bfloat16),
                        pltpu.VMEM((Ho * Wo, Cout), jnp.float32)],
        compiler_params=_cp("parallel", "arbitrary"),
    )(xp, xp, xp, wm, b)


def _conv_s3_kernel(x0_ref, x1_ref, x2_ref, x3_ref, w_ref, b_ref,
                    z_ref, s_ref, sq_ref, patch_ref, acc_ref, *, Ho, Wo, Cin, Cout):
    @pl.when(pl.program_id(1) == 0)
    def _():
        s_ref[...] = jnp.zeros_like(s_ref)
        sq_ref[...] = jnp.zeros_like(sq_ref)

    M = Ho * Wo
    refs = (x0_ref, x1_ref, x2_ref, x3_ref)
    for od in range(2):
        acc_ref[...] = jnp.broadcast_to(b_ref[...], acc_ref.shape)
        for kd in range(3):
            x_ref = refs[od + kd]
            for kh in range(3):
                for kw in range(3):
                    tap = x_ref[0, 0, kh:kh + Ho, kw:kw + Wo, :]
                    c0 = (kh * 3 + kw) * Cin
                    patch_ref[:, c0:c0 + Cin] = tap.reshape(M, Cin)
            acc_ref[...] += jnp.dot(patch_ref[...], w_ref[kd],
                                    preferred_element_type=jnp.float32)
        acc = acc_ref[...]
        z_ref[0, od] = acc.reshape(Ho, Wo, Cout).astype(jnp.bfloat16)
        zg = acc.reshape(-1, 8, Cout)
        s_ref[...] += jnp.sum(zg, axis=0)
        sq_ref[...] += jnp.sum(zg * zg, axis=0)


def _conv_s3(xp, wm, b, *, Do, Ho, Wo, Cin, Cout):
    N = xp.shape[0]
    _, _, Hp, Wp, _ = xp.shape
    Dh = Do // 2
    half = (N * Dh) // 2

    def xmapj(j):
        def m(c, i):
            g = c * half + i
            return (g // Dh, 2 * (g % Dh) + j, 0, 0, 0)
        return m

    def zmap(c, i):
        g = c * half + i
        return (g // Dh, g % Dh, 0, 0, 0)

    kern = functools.partial(_conv_s3_kernel, Ho=Ho, Wo=Wo, Cin=Cin, Cout=Cout)
    return pl.pallas_call(
        kern,
        out_shape=(jax.ShapeDtypeStruct((N, Do, Ho, Wo, Cout), jnp.bfloat16),
                   jax.ShapeDtypeStruct((16, Cout), jnp.float32),
                   jax.ShapeDtypeStruct((16, Cout), jnp.float32)),
        grid=(2, half),
        in_specs=[pl.BlockSpec((1, 1, Hp, Wp, Cin), xmapj(0)),
                  pl.BlockSpec((1, 1, Hp, Wp, Cin), xmapj(1)),
                  pl.BlockSpec((1, 1, Hp, Wp, Cin), xmapj(2)),
                  pl.BlockSpec((1, 1, Hp, Wp, Cin), xmapj(3)),
                  pl.BlockSpec((3, 9 * Cin, Cout), lambda c, i: (0, 0, 0)),
                  pl.BlockSpec((1, Cout), lambda c, i: (0, 0))],
        out_specs=(pl.BlockSpec((1, 2, Ho, Wo, Cout), zmap),
                   pl.BlockSpec((8, Cout), lambda c, i: (c, 0)),
                   pl.BlockSpec((8, Cout), lambda c, i: (c, 0))),
        scratch_shapes=[pltpu.VMEM((Ho * Wo, 9 * Cin), jnp.bfloat16),
                        pltpu.VMEM((Ho * Wo, Cout), jnp.float32)],
        compiler_params=_cp("parallel", "arbitrary"),
    )(xp, xp, xp, xp, wm, b)


def _s2_post_kernel(z_ref, sc_ref, sh_ref, o_ref, *, slope, dpad):
    g = pl.program_id(0) * pl.num_programs(1) + pl.program_id(1)
    dp = g % dpad
    interior = (dp >= 1) & (dp <= dpad - 2)

    @pl.when(interior)
    def _():
        y = z_ref[0, 0].astype(jnp.float32) * sc_ref[0] + sh_ref[0]
        y = jnp.where(y >= 0, y, slope * y).astype(jnp.bfloat16)
        H, W, C = y.shape
        zr = jnp.zeros((1, W, C), jnp.bfloat16)
        y = jnp.concatenate([zr, y, zr], axis=0)
        zc = jnp.zeros((H + 2, 1, C), jnp.bfloat16)
        o_ref[0, 0] = jnp.concatenate([zc, y, zc], axis=1)

    @pl.when(jnp.logical_not(interior))
    def _():
        o_ref[...] = jnp.zeros_like(o_ref)


def _s2_post(z5, sc, sh, slope):
    N, D, H, W, C = z5.shape
    dpad = D + 2
    half = (N * dpad) // 2

    def zmap(c, i):
        g = c * half + i
        return (g // dpad, jnp.clip(g % dpad - 1, 0, D - 1), 0, 0, 0)

    def omap(c, i):
        g = c * half + i
        return (g // dpad, g % dpad, 0, 0, 0)

    return pl.pallas_call(
        functools.partial(_s2_post_kernel, slope=slope, dpad=dpad),
        out_shape=jax.ShapeDtypeStruct((N, dpad, H + 2, W + 2, C), jnp.bfloat16),
        grid=(2, half),
        in_specs=[pl.BlockSpec((1, 1, H, W, C), zmap),
                  pl.BlockSpec((1, C), lambda c, i: (0, 0)),
                  pl.BlockSpec((1, C), lambda c, i: (0, 0))],
        out_specs=pl.BlockSpec((1, 1, H + 2, W + 2, C), omap),
        compiler_params=_cp("parallel", "arbitrary"),
    )(z5, sc, sh)


def _tail_kernel(z_ref, sc_ref, sh_ref, w_ref, o_ref, *, slope):
    @pl.when(pl.program_id(1) == 0)
    def _():
        o_ref[...] = jnp.zeros_like(o_ref)

    def act(v):
        u = v.astype(jnp.float32) * sc_ref[0] + sh_ref[0]
        return jnp.where(u >= 0, u, slope * u)

    y = act(z_ref[0, 0]) + act(z_ref[0, 1])
    H, W, C = y.shape
    y = y.reshape(H // 2, 2, W, C)
    y = (y[:, 0] + y[:, 1]) * 0.25
    s = jnp.sum(y * w_ref[0])
    o_ref[...] += jnp.full(o_ref.shape, s, jnp.float32)


def _tail(z3, sc, sh, wv, slope):
    N, D, H, W, C = z3.shape
    D2 = D // 2
    out = pl.pallas_call(
        functools.partial(_tail_kernel, slope=slope),
        out_shape=jax.ShapeDtypeStruct((N, 8, 128), jnp.float32),
        grid=(N, D2),
        in_specs=[pl.BlockSpec((1, 2, H, W, C), lambda n, d: (n, d, 0, 0, 0)),
                  pl.BlockSpec((1, C), lambda n, d: (0, 0)),
                  pl.BlockSpec((1, C), lambda n, d: (0, 0)),
                  pl.BlockSpec((1, H // 2, W, C), lambda n, d: (d, 0, 0, 0))],
        out_specs=pl.BlockSpec((1, 8, 128), lambda n, d: (n, 0, 0)),
        compiler_params=_cp("parallel", "arbitrary"),
    )(z3, sc, sh, wv)
    return out[:, 0, :1]


def _im2colT_c1_bf16(x, stride):
    assert stride == 2
    N, D, H, W = x.shape
    Do, Ho, Wo = D // 2, H // 2, W // 2
    xp = jnp.pad(x, ((0, 0), (1, 1), (1, 1), (1, 1)))
    xps = xp.reshape(N, D + 2, Ho + 1, 2, Wo + 1, 2)
    xps = xps.transpose(0, 3, 5, 1, 2, 4)
    M = N * Do * Ho * Wo
    rows = []
    for kd in range(3):
        for kh in range(3):
            for kw in range(3):
                ph, h0 = kh % 2, kh // 2
                pw, w0 = kw % 2, kw // 2
                sl = xps[:, ph, pw, kd:kd + 2 * Do - 1:2,
                         h0:h0 + Ho, w0:w0 + Wo]
                rows.append(sl.astype(jnp.bfloat16).reshape(1, M))
    return jnp.concatenate(rows, axis=0), (Do, Ho, Wo)


def _affine(s, sq, count, gamma, beta, eps):
    s = jnp.sum(s, axis=0)
    sq = jnp.sum(sq, axis=0)
    mean = s / count
    var = sq / count - mean * mean
    scale = gamma / jnp.sqrt(var + eps)
    shift = beta - mean * scale
    C = scale.shape[0]
    return scale.reshape(1, C), shift.reshape(1, C)


def kernel(x, w1, b1, g1, be1, w2, b2, g2, be2, w3, b3, g3, be3, wl, bl):
    eps, slope = 1e-5, 0.2
    N = x.shape[0]
    bf = jnp.bfloat16

    pT, (Do1, Ho1, Wo1) = _im2colT_c1_bf16(x[:, 0], stride=2)
    M1 = pT.shape[1]
    G, rs = _s1_gram(pT, tm=16384)
    w1f = w1.reshape(32, 27).T
    wr = w1f.T @ rs
    s1 = wr + M1 * b1
    sq1 = jnp.einsum("kc,kl,lc->c", w1f, G, w1f) + 2.0 * b1 * wr + M1 * b1 * b1
    mean1 = s1 / M1
    var1 = sq1 / M1 - mean1 * mean1
    sc1 = g1 / jnp.sqrt(var1 + eps)
    sh1 = be1 - mean1 * sc1
    xph2 = _s1_post(pT, w1f.astype(bf), b1.reshape(1, 32),
                    sc1.reshape(1, 32), sh1.reshape(1, 32), slope, N, Do1)

    w2m = w2.transpose(2, 3, 4, 1, 0).reshape(3, 9 * 32, 64).astype(bf)
    Do2, Ho2, Wo2 = Do1 // 2, Ho1 // 2, Wo1 // 2
    z2, s2, sq2 = _conv_s2(xph2, w2m, b2.reshape(1, 64),
                           Do=Do2, Ho=Ho2, Wo=Wo2, Cin=32, Cout=64)
    sc2, sh2 = _affine(s2, sq2, N * Do2 * Ho2 * Wo2, g2, be2, eps)
    x3 = _s2_post(z2, sc2, sh2, slope)

    w3m = w3.transpose(2, 3, 4, 1, 0).reshape(3, 9 * 64, 128).astype(bf)
    z3, s3, sq3 = _conv_s3(x3, w3m, b3.reshape(1, 128),
                           Do=Do2, Ho=Ho2, Wo=Wo2, Cin=64, Cout=128)
    sc3, sh3 = _affine(s3, sq3, N * Do2 * Ho2 * Wo2, g3, be3, eps)

    D2, H2, W2 = Do2 // 2, Ho2 // 2, Wo2 // 2
    wv = wl.reshape(128, D2, H2, W2)
    wv = jnp.repeat(wv, 2, axis=3) * 0.5
    wv = wv.transpose(1, 2, 3, 0)
    out = _tail(z3, sc3, sh3, wv, slope)
    return out + bl[None, :]


# speedup vs baseline: 22.2196x; 22.2196x over previous
import functools

import jax
import jax.numpy as jnp
import numpy as np
from jax.experimental import pallas as pl
from jax.experimental.pallas import tpu as pltpu

_VMEM = 56 * 1024 * 1024


def _cp(*sem):
    return pltpu.CompilerParams(dimension_semantics=sem,
                                vmem_limit_bytes=_VMEM)


def _phase_split_kernel(x_ref, o_ref):
    y = x_ref[0, 0]
    H2, W2 = y.shape[0] // 2, y.shape[1] // 2
    yv = y.reshape(H2, 2, W2, 2)
    planes = [[yv[:, ph, :, pw].astype(jnp.bfloat16) for pw in range(2)]
              for ph in range(2)]
    o_ref[0, :, :, 0] = jnp.stack([jnp.stack(p, axis=0) for p in planes], axis=0)


def _phase_split(xp):
    N, Dp, Hp, Wp = xp.shape
    half = (N * Dp) // 2

    def imap(c, i):
        g = c * half + i
        return (g // Dp, g % Dp, 0, 0)

    def omap(c, i):
        g = c * half + i
        return (g // Dp, 0, 0, g % Dp, 0, 0)

    return pl.pallas_call(
        _phase_split_kernel,
        out_shape=jax.ShapeDtypeStruct((N, 2, 2, Dp, Hp // 2, Wp // 2),
                                       jnp.bfloat16),
        grid=(2, half),
        in_specs=[pl.BlockSpec((1, 1, Hp, Wp), imap)],
        out_specs=pl.BlockSpec((1, 2, 2, 1, Hp // 2, Wp // 2), omap),
        compiler_params=_cp("parallel", "arbitrary"),
    )(xp)


def _s1_gram_kernel(p_ref, g_ref):
    @pl.when(pl.program_id(1) == 0)
    def _():
        g_ref[...] = jnp.zeros_like(g_ref)

    a = p_ref[...]
    G = jax.lax.dot_general(a, a, (((1,), (1,)), ((), ())),
                            preferred_element_type=jnp.float32)
    rs = jnp.sum(a.astype(jnp.float32), axis=1, keepdims=True)
    g_ref[0:27, 0:28] += jnp.concatenate([G, rs], axis=1)


def _s1_gram(pT, tm):
    K, M = pT.shape
    half = M // (2 * tm)
    out = pl.pallas_call(
        _s1_gram_kernel,
        out_shape=jax.ShapeDtypeStruct((64, 128), jnp.float32),
        grid=(2, half),
        in_specs=[pl.BlockSpec((K, tm), lambda c, i: (0, c * (M // (2 * tm)) + i))],
        out_specs=pl.BlockSpec((32, 128), lambda c, i: (c, 0)),
        compiler_params=_cp("parallel", "arbitrary"),
    )(pT)
    s = out[:32] + out[32:]
    return s[0:27, 0:27], s[0:27, 27]


def _s1_post_kernel(p_ref, w_ref, b_ref, sc_ref, sh_ref, o_ref, *, slope, dpad):
    g = pl.program_id(0) * pl.num_programs(1) + pl.program_id(1)
    dp = g % dpad
    interior = (dp >= 1) & (dp <= dpad - 2)

    @pl.when(interior)
    def _():
        z = jax.lax.dot_general(p_ref[...], w_ref[...], (((0,), (0,)), ((), ())),
                                preferred_element_type=jnp.float32)
        z = z + b_ref[...]
        y = z * sc_ref[...] + sh_ref[...]
        y = jnp.where(y >= 0, y, slope * y).astype(jnp.bfloat16)
        C = y.shape[1]
        y = y.reshape(128, 128, C)
        yr = y.reshape(64, 2, 64, 2, C)
        planes = []
        for ph in range(2):
            for pw in range(2):
                q = yr[:, 1 - ph, :, 1 - pw, :]
                zr = jnp.zeros((1, 64, C), jnp.bfloat16)
                q = jnp.concatenate([zr, q] if ph == 0 else [q, zr], axis=0)
                zc = jnp.zeros((65, 1, C), jnp.bfloat16)
                q = jnp.concatenate([zc, q] if pw == 0 else [q, zc], axis=1)
                planes.append(q)
        o_ref[0, :, 0] = jnp.stack(planes, axis=0)

    @pl.when(jnp.logical_not(interior))
    def _():
        o_ref[...] = jnp.zeros_like(o_ref)


def _s1_post(pT, wm, b, sc, sh, slope, N, D):
    dpad = D + 2
    G = N * dpad
    half = G // 2
    mblk = 128 * 128

    def pmap(c, i):
        g = c * half + i
        n, dp = g // dpad, g % dpad
        return (0, jnp.clip(n * D + dp - 1, 0, N * D - 1))

    def omap(c, i):
        g = c * half + i
        return (g // dpad, 0, g % dpad, 0, 0, 0)

    return pl.pallas_call(
        functools.partial(_s1_post_kernel, slope=slope, dpad=dpad),
        out_shape=jax.ShapeDtypeStruct((N, 4, dpad, 65, 65, 32), jnp.bfloat16),
        grid=(2, half),
        in_specs=[pl.BlockSpec((27, mblk), pmap),
                  pl.BlockSpec((27, 32), lambda c, i: (0, 0)),
                  pl.BlockSpec((1, 32), lambda c, i: (0, 0)),
                  pl.BlockSpec((1, 32), lambda c, i: (0, 0)),
                  pl.BlockSpec((1, 32), lambda c, i: (0, 0))],
        out_specs=pl.BlockSpec((1, 4, 1, 65, 65, 32), omap),
        compiler_params=_cp("parallel", "arbitrary"),
    )(pT, wm, b, sc, sh)


def _conv_s2_kernel(x0_ref, x1_ref, x2_ref, w_ref, b_ref, z_ref, s_ref, sq_ref,
                    patch_ref, acc_ref, *, Ho, Wo, Cin, Cout):
    @pl.when(pl.program_id(1) == 0)
    def _():
        s_ref[...] = jnp.zeros_like(s_ref)
        sq_ref[...] = jnp.zeros_like(sq_ref)

    acc_ref[...] = jnp.broadcast_to(b_ref[...], acc_ref.shape)
    M = Ho * Wo
    for kd, x_ref in enumerate((x0_ref, x1_ref, x2_ref)):
        for kh in range(3):
            for kw in range(3):
                p, h0, w0 = 2 * (kh % 2) + (kw % 2), kh // 2, kw // 2
                tap = x_ref[0, p, 0, h0:h0 + Ho, w0:w0 + Wo, :]
                c0 = (kh * 3 + kw) * Cin
                patch_ref[:, c0:c0 + Cin] = tap.reshape(M, Cin)
        acc_ref[...] += jnp.dot(patch_ref[...], w_ref[kd],
                                preferred_element_type=jnp.float32)

    acc = acc_ref[...]
    z_ref[0, 0] = acc.reshape(Ho, Wo, Cout).astype(jnp.bfloat16)
    zg = acc.reshape(-1, 8, Cout)
    s_ref[...] += jnp.sum(zg, axis=0)
    sq_ref[...] += jnp.sum(zg * zg, axis=0)


def _conv_s2(xp, wm, b, *, Do, Ho, Wo, Cin, Cout):
    N = xp.shape[0]
    _, _, _, Hp, Wp, _ = xp.shape
    half = (N * Do) // 2

    def xmapj(j):
        def m(c, i):
            g = c * half + i
            return (g // Do, 0, 2 * (g % Do) + j, 0, 0, 0)
        return m

    def zmap(c, i):
        g = c * half + i
        return (g // Do, g % Do, 0, 0, 0)

    kern = functools.partial(_conv_s2_kernel, Ho=Ho, Wo=Wo, Cin=Cin, Cout=Cout)
    return pl.pallas_call(
        kern,
        out_shape=(jax.ShapeDtypeStruct((N, Do, Ho, Wo, Cout), jnp.bfloat16),
                   jax.ShapeDtypeStruct((16, Cout), jnp.float32),
                   jax.ShapeDtypeStruct((16, Cout), jnp.float32)),
        grid=(2, half),
        in_specs=[pl.BlockSpec((1, 4, 1, Hp, Wp, Cin), xmapj(0)),
                  pl.BlockSpec((1, 4, 1, Hp, Wp, Cin), xmapj(1)),
                  pl.BlockSpec((1, 4, 1, Hp, Wp, Cin), xmapj(2)),
                  pl.BlockSpec((3, 9 * Cin, Cout), lambda c, i: (0, 0, 0)),
                  pl.BlockSpec((1, Cout), lambda c, i: (0, 0))],
        out_specs=(pl.BlockSpec((1, 1, Ho, Wo, Cout), zmap),
                   pl.BlockSpec((8, Cout), lambda c, i: (c, 0)),
                   pl.BlockSpec((8, Cout), lambda c, i: (c, 0))),
        scratch_shapes=[pltpu.VMEM((Ho * Wo, 9 * Cin), jnp.bfloat16),
                        pltpu.VMEM((Ho * Wo, Cout), jnp.float32)],
        compiler_params=_cp("parallel", "arbitrary"),
    )(xp, xp, xp, wm, b)


def _conv_s3_kernel(x0_ref, x1_ref, x2_ref, x3_ref, w_ref, b_ref,
                    z_ref, s_ref, sq_ref, patch_ref, acc_ref, *, Ho, Wo, Cin, Cout):
    @pl.when(pl.program_id(1) == 0)
    def _():
        s_ref[...] = jnp.zeros_like(s_ref)
        sq_ref[...] = jnp.zeros_like(sq_ref)

    M = Ho * Wo
    refs = (x0_ref, x1_ref, x2_ref, x3_ref)
    for od in range(2):
        acc_ref[...] = jnp.broadcast_to(b_ref[...], acc_ref.shape)
        for kd in range(3):
            x_ref = refs[od + kd]
            for kh in range(3):
                for kw in range(3):
                    tap = x_ref[0, 0, kh:kh + Ho, kw:kw + Wo, :]
                    c0 = (kh * 3 + kw) * Cin
                    patch_ref[:, c0:c0 + Cin] = tap.reshape(M, Cin)
            acc_ref[...] += jnp.dot(patch_ref[...], w_ref[kd],
                                    preferred_element_type=jnp.float32)
        acc = acc_ref[...]
        z_ref[0, od] = acc.reshape(Ho, Wo, Cout).astype(jnp.bfloat16)
        zg = acc.reshape(-1, 8, Cout)
        s_ref[...] += jnp.sum(zg, axis=0)
        sq_ref[...] += jnp.sum(zg * zg, axis=0)


def _conv_s3(xp, wm, b, *, Do, Ho, Wo, Cin, Cout):
    N = xp.shape[0]
    _, _, Hp, Wp, _ = xp.shape
    Dh = Do // 2
    half = (N * Dh) // 2

    def xmapj(j):
        def m(c, i):
            g = c * half + i
            return (g // Dh, 2 * (g % Dh) + j, 0, 0, 0)
        return m

    def zmap(c, i):
        g = c * half + i
        return (g // Dh, g % Dh, 0, 0, 0)

    kern = functools.partial(_conv_s3_kernel, Ho=Ho, Wo=Wo, Cin=Cin, Cout=Cout)
    return pl.pallas_call(
        kern,
        out_shape=(jax.ShapeDtypeStruct((N, Do, Ho, Wo, Cout), jnp.bfloat16),
                   jax.ShapeDtypeStruct((16, Cout), jnp.float32),
                   jax.ShapeDtypeStruct((16, Cout), jnp.float32)),
        grid=(2, half),
        in_specs=[pl.BlockSpec((1, 1, Hp, Wp, Cin), xmapj(0)),
                  pl.BlockSpec((1, 1, Hp, Wp, Cin), xmapj(1)),
                  pl.BlockSpec((1, 1, Hp, Wp, Cin), xmapj(2)),
                  pl.BlockSpec((1, 1, Hp, Wp, Cin), xmapj(3)),
                  pl.BlockSpec((3, 9 * Cin, Cout), lambda c, i: (0, 0, 0)),
                  pl.BlockSpec((1, Cout), lambda c, i: (0, 0))],
        out_specs=(pl.BlockSpec((1, 2, Ho, Wo, Cout), zmap),
                   pl.BlockSpec((8, Cout), lambda c, i: (c, 0)),
                   pl.BlockSpec((8, Cout), lambda c, i: (c, 0))),
        scratch_shapes=[pltpu.VMEM((Ho * Wo, 9 * Cin), jnp.bfloat16),
                        pltpu.VMEM((Ho * Wo, Cout), jnp.float32)],
        compiler_params=_cp("parallel", "arbitrary"),
    )(xp, xp, xp, xp, wm, b)


def _s2_post_kernel(z_ref, sc_ref, sh_ref, o_ref, *, slope, dpad):
    g = pl.program_id(0) * pl.num_programs(1) + pl.program_id(1)
    dp = g % dpad
    interior = (dp >= 1) & (dp <= dpad - 2)

    @pl.when(interior)
    def _():
        y = z_ref[0, 0].astype(jnp.float32) * sc_ref[0] + sh_ref[0]
        y = jnp.where(y >= 0, y, slope * y).astype(jnp.bfloat16)
        H, W, C = y.shape
        zr = jnp.zeros((1, W, C), jnp.bfloat16)
        y = jnp.concatenate([zr, y, zr], axis=0)
        zc = jnp.zeros((H + 2, 1, C), jnp.bfloat16)
        o_ref[0, 0] = jnp.concatenate([zc, y, zc], axis=1)

    @pl.when(jnp.logical_not(interior))
    def _():
        o_ref[...] = jnp.zeros_like(o_ref)


def _s2_post(z5, sc, sh, slope):
    N, D, H, W, C = z5.shape
    dpad = D + 2
    half = (N * dpad) // 2

    def zmap(c, i):
        g = c * half + i
        return (g // dpad, jnp.clip(g % dpad - 1, 0, D - 1), 0, 0, 0)

    def omap(c, i):
        g = c * half + i
        return (g // dpad, g % dpad, 0, 0, 0)

    return pl.pallas_call(
        functools.partial(_s2_post_kernel, slope=slope, dpad=dpad),
        out_shape=jax.ShapeDtypeStruct((N, dpad, H + 2, W + 2, C), jnp.bfloat16),
        grid=(2, half),
        in_specs=[pl.BlockSpec((1, 1, H, W, C), zmap),
                  pl.BlockSpec((1, C), lambda c, i: (0, 0)),
                  pl.BlockSpec((1, C), lambda c, i: (0, 0))],
        out_specs=pl.BlockSpec((1, 1, H + 2, W + 2, C), omap),
        compiler_params=_cp("parallel", "arbitrary"),
    )(z5, sc, sh)


def _tail_kernel(z_ref, sc_ref, sh_ref, w_ref, o_ref, *, slope):
    @pl.when(pl.program_id(1) == 0)
    def _():
        o_ref[...] = jnp.zeros_like(o_ref)

    def act(v):
        u = v.astype(jnp.float32) * sc_ref[0] + sh_ref[0]
        return jnp.where(u >= 0, u, slope * u)

    y = act(z_ref[0, 0]) + act(z_ref[0, 1])
    H, W, C = y.shape
    y = y.reshape(H // 2, 2, W, C)
    y = (y[:, 0] + y[:, 1]) * 0.25
    s = jnp.sum(y * w_ref[0])
    o_ref[...] += jnp.full(o_ref.shape, s, jnp.float32)


def _tail(z3, sc, sh, wv, slope):
    N, D, H, W, C = z3.shape
    D2 = D // 2
    out = pl.pallas_call(
        functools.partial(_tail_kernel, slope=slope),
        out_shape=jax.ShapeDtypeStruct((N, 8, 128), jnp.float32),
        grid=(N, D2),
        in_specs=[pl.BlockSpec((1, 2, H, W, C), lambda n, d: (n, d, 0, 0, 0)),
                  pl.BlockSpec((1, C), lambda n, d: (0, 0)),
                  pl.BlockSpec((1, C), lambda n, d: (0, 0)),
                  pl.BlockSpec((1, H // 2, W, C), lambda n, d: (d, 0, 0, 0))],
        out_specs=pl.BlockSpec((1, 8, 128), lambda n, d: (n, 0, 0)),
        compiler_params=_cp("parallel", "arbitrary"),
    )(z3, sc, sh, wv)
    return out[:, 0, :1]


def _im2colT_c1_bf16(x, stride):
    assert stride == 2
    N, D, H, W = x.shape
    Do, Ho, Wo = D // 2, H // 2, W // 2
    xp = jnp.pad(x, ((0, 0), (1, 1), (1, 1), (1, 1)))
    xps = _phase_split(xp)
    M = N * Do * Ho * Wo
    rows = []
    for kd in range(3):
        for kh in range(3):
            for kw in range(3):
                ph, h0 = kh % 2, kh // 2
                pw, w0 = kw % 2, kw // 2
                sl = xps[:, ph, pw, kd:kd + 2 * Do - 1:2,
                         h0:h0 + Ho, w0:w0 + Wo]
                rows.append(sl.reshape(1, M))
    return jnp.concatenate(rows, axis=0), (Do, Ho, Wo)


def _affine(s, sq, count, gamma, beta, eps):
    s = jnp.sum(s, axis=0)
    sq = jnp.sum(sq, axis=0)
    mean = s / count
    var = sq / count - mean * mean
    scale = gamma / jnp.sqrt(var + eps)
    shift = beta - mean * scale
    C = scale.shape[0]
    return scale.reshape(1, C), shift.reshape(1, C)


def kernel(x, w1, b1, g1, be1, w2, b2, g2, be2, w3, b3, g3, be3, wl, bl):
    eps, slope = 1e-5, 0.2
    N = x.shape[0]
    bf = jnp.bfloat16

    pT, (Do1, Ho1, Wo1) = _im2colT_c1_bf16(x[:, 0], stride=2)
    M1 = pT.shape[1]
    G, rs = _s1_gram(pT, tm=16384)
    w1f = w1.reshape(32, 27).T
    wr = w1f.T @ rs
    s1 = wr + M1 * b1
    sq1 = jnp.einsum("kc,kl,lc->c", w1f, G, w1f) + 2.0 * b1 * wr + M1 * b1 * b1
    mean1 = s1 / M1
    var1 = sq1 / M1 - mean1 * mean1
    sc1 = g1 / jnp.sqrt(var1 + eps)
    sh1 = be1 - mean1 * sc1
    xph2 = _s1_post(pT, w1f.astype(bf), b1.reshape(1, 32),
                    sc1.reshape(1, 32), sh1.reshape(1, 32), slope, N, Do1)

    w2m = w2.transpose(2, 3, 4, 1, 0).reshape(3, 9 * 32, 64).astype(bf)
    Do2, Ho2, Wo2 = Do1 // 2, Ho1 // 2, Wo1 // 2
    z2, s2, sq2 = _conv_s2(xph2, w2m, b2.reshape(1, 64),
                           Do=Do2, Ho=Ho2, Wo=Wo2, Cin=32, Cout=64)
    sc2, sh2 = _affine(s2, sq2, N * Do2 * Ho2 * Wo2, g2, be2, eps)
    x3 = _s2_post(z2, sc2, sh2, slope)

    w3m = w3.transpose(2, 3, 4, 1, 0).reshape(3, 9 * 64, 128).astype(bf)
    z3, s3, sq3 = _conv_s3(x3, w3m, b3.reshape(1, 128),
                           Do=Do2, Ho=Ho2, Wo=Wo2, Cin=64, Cout=128)
    sc3, sh3 = _affine(s3, sq3, N * Do2 * Ho2 * Wo2, g3, be3, eps)

    D2, H2, W2 = Do2 // 2, Ho2 // 2, Wo2 // 2
    wv = wl.reshape(128, D2, H2, W2)
    wv = jnp.repeat(wv, 2, axis=3) * 0.5
    wv = wv.transpose(1, 2, 3, 0)
    out = _tail(z3, sc3, sh3, wv, slope)
    return out + bl[None, :]


# speedup vs baseline: 24.0228x; 1.0812x over previous
import functools

import jax
import jax.numpy as jnp
import numpy as np
from jax.experimental import pallas as pl
from jax.experimental.pallas import tpu as pltpu

_VMEM = 56 * 1024 * 1024


def _cp(*sem):
    return pltpu.CompilerParams(dimension_semantics=sem,
                                vmem_limit_bytes=_VMEM)


def _s1_gram_kernel(p_ref, g_ref):
    @pl.when(pl.program_id(1) == 0)
    def _():
        g_ref[...] = jnp.zeros_like(g_ref)

    a = p_ref[...]
    G = jax.lax.dot_general(a, a, (((1,), (1,)), ((), ())),
                            preferred_element_type=jnp.float32)
    rs = jnp.sum(a.astype(jnp.float32), axis=1, keepdims=True)
    g_ref[0:27, 0:28] += jnp.concatenate([G, rs], axis=1)


def _s1_gram(pT, tm):
    K, M = pT.shape
    half = M // (2 * tm)
    out = pl.pallas_call(
        _s1_gram_kernel,
        out_shape=jax.ShapeDtypeStruct((64, 128), jnp.float32),
        grid=(2, half),
        in_specs=[pl.BlockSpec((K, tm), lambda c, i: (0, c * (M // (2 * tm)) + i))],
        out_specs=pl.BlockSpec((32, 128), lambda c, i: (c, 0)),
        compiler_params=_cp("parallel", "arbitrary"),
    )(pT)
    s = out[:32] + out[32:]
    return s[0:27, 0:27], s[0:27, 27]


def _s1_post_kernel(p_ref, w_ref, b_ref, sc_ref, sh_ref, o_ref, *, slope, dpad):
    g = pl.program_id(0) * pl.num_programs(1) + pl.program_id(1)
    dp = g % dpad
    interior = (dp >= 1) & (dp <= dpad - 2)

    @pl.when(interior)
    def _():
        z = jax.lax.dot_general(p_ref[...], w_ref[...], (((0,), (0,)), ((), ())),
                                preferred_element_type=jnp.float32)
        z = z + b_ref[...]
        y = z * sc_ref[...] + sh_ref[...]
        y = jnp.where(y >= 0, y, slope * y).astype(jnp.bfloat16)
        C = y.shape[1]
        yr = y.reshape(64, 2, 64, 2, C)
        planes = []
        for ph in range(2):
            for pw in range(2):
                q = yr[:, 1 - ph, :, 1 - pw, :]
                zr = jnp.zeros((1, 64, C), jnp.bfloat16)
                q = jnp.concatenate([zr, q] if ph == 0 else [q, zr], axis=0)
                zc = jnp.zeros((65, 1, C), jnp.bfloat16)
                q = jnp.concatenate([zc, q] if pw == 0 else [q, zc], axis=1)
                planes.append(q)
        o_ref[0, :, 0] = jnp.stack(planes, axis=0)

    @pl.when(jnp.logical_not(interior))
    def _():
        o_ref[...] = jnp.zeros_like(o_ref)


def _s1_post(pT, wm, b, sc, sh, slope, N, D):
    dpad = D + 2
    G = N * dpad
    half = G // 2
    mblk = 128 * 128

    def pmap(c, i):
        g = c * half + i
        n, dp = g // dpad, g % dpad
        return (0, jnp.clip(n * D + dp - 1, 0, N * D - 1))

    def omap(c, i):
        g = c * half + i
        return (g // dpad, 0, g % dpad, 0, 0, 0)

    return pl.pallas_call(
        functools.partial(_s1_post_kernel, slope=slope, dpad=dpad),
        out_shape=jax.ShapeDtypeStruct((N, 4, dpad, 65, 65, 32), jnp.bfloat16),
        grid=(2, half),
        in_specs=[pl.BlockSpec((27, mblk), pmap),
                  pl.BlockSpec((27, 32), lambda c, i: (0, 0)),
                  pl.BlockSpec((1, 32), lambda c, i: (0, 0)),
                  pl.BlockSpec((1, 32), lambda c, i: (0, 0)),
                  pl.BlockSpec((1, 32), lambda c, i: (0, 0))],
        out_specs=pl.BlockSpec((1, 4, 1, 65, 65, 32), omap),
        compiler_params=_cp("parallel", "arbitrary"),
    )(pT, wm, b, sc, sh)


def _conv_s2_kernel(x0_ref, x1_ref, x2_ref, w_ref, b_ref, z_ref, s_ref, sq_ref,
                    patch_ref, acc_ref, *, Ho, Wo, Cin, Cout):
    @pl.when(pl.program_id(1) == 0)
    def _():
        s_ref[...] = jnp.zeros_like(s_ref)
        sq_ref[...] = jnp.zeros_like(sq_ref)

    acc_ref[...] = jnp.broadcast_to(b_ref[...], acc_ref.shape)
    M = Ho * Wo
    for kd, x_ref in enumerate((x0_ref, x1_ref, x2_ref)):
        for kh in range(3):
            for kw in range(3):
                p, h0, w0 = 2 * (kh % 2) + (kw % 2), kh // 2, kw // 2
                tap = x_ref[0, p, 0, h0:h0 + Ho, w0:w0 + Wo, :]
                c0 = (kh * 3 + kw) * Cin
                patch_ref[:, c0:c0 + Cin] = tap.reshape(M, Cin)
        acc_ref[...] += jnp.dot(patch_ref[...], w_ref[kd],
                                preferred_element_type=jnp.float32)

    acc = acc_ref[...]
    z_ref[0, 0] = acc.reshape(Ho, Wo, Cout)
    zg = acc.reshape(-1, 8, Cout)
    s_ref[...] += jnp.sum(zg, axis=0)
    sq_ref[...] += jnp.sum(zg * zg, axis=0)


def _conv_s2(xp, wm, b, *, Do, Ho, Wo, Cin, Cout):
    N = xp.shape[0]
    _, _, _, Hp, Wp, _ = xp.shape
    half = (N * Do) // 2

    def xmapj(j):
        def m(c, i):
            g = c * half + i
            return (g // Do, 0, 2 * (g % Do) + j, 0, 0, 0)
        return m

    def zmap(c, i):
        g = c * half + i
        return (g // Do, g % Do, 0, 0, 0)

    kern = functools.partial(_conv_s2_kernel, Ho=Ho, Wo=Wo, Cin=Cin, Cout=Cout)
    return pl.pallas_call(
        kern,
        out_shape=(jax.ShapeDtypeStruct((N, Do, Ho, Wo, Cout), jnp.float32),
                   jax.ShapeDtypeStruct((16, Cout), jnp.float32),
                   jax.ShapeDtypeStruct((16, Cout), jnp.float32)),
        grid=(2, half),
        in_specs=[pl.BlockSpec((1, 4, 1, Hp, Wp, Cin), xmapj(0)),
                  pl.BlockSpec((1, 4, 1, Hp, Wp, Cin), xmapj(1)),
                  pl.BlockSpec((1, 4, 1, Hp, Wp, Cin), xmapj(2)),
                  pl.BlockSpec((3, 9 * Cin, Cout), lambda c, i: (0, 0, 0)),
                  pl.BlockSpec((1, Cout), lambda c, i: (0, 0))],
        out_specs=(pl.BlockSpec((1, 1, Ho, Wo, Cout), zmap),
                   pl.BlockSpec((8, Cout), lambda c, i: (c, 0)),
                   pl.BlockSpec((8, Cout), lambda c, i: (c, 0))),
        scratch_shapes=[pltpu.VMEM((Ho * Wo, 9 * Cin), jnp.bfloat16),
                        pltpu.VMEM((Ho * Wo, Cout), jnp.float32)],
        compiler_params=_cp("parallel", "arbitrary"),
    )(xp, xp, xp, wm, b)


def _conv_s3_kernel(x0_ref, x1_ref, x2_ref, x3_ref, w_ref, b_ref,
                    z_ref, s_ref, sq_ref, patch_ref, acc_ref, *, Ho, Wo, Cin, Cout):
    @pl.when(pl.program_id(1) == 0)
    def _():
        s_ref[...] = jnp.zeros_like(s_ref)
        sq_ref[...] = jnp.zeros_like(sq_ref)

    M = Ho * Wo
    refs = (x0_ref, x1_ref, x2_ref, x3_ref)
    for od in range(2):
        acc_ref[...] = jnp.broadcast_to(b_ref[...], acc_ref.shape)
        for kd in range(3):
            x_ref = refs[od + kd]
            for kh in range(3):
                for kw in range(3):
                    tap = x_ref[0, 0, kh:kh + Ho, kw:kw + Wo, :]
                    c0 = (kh * 3 + kw) * Cin
                    patch_ref[:, c0:c0 + Cin] = tap.reshape(M, Cin)
            acc_ref[...] += jnp.dot(patch_ref[...], w_ref[kd],
                                    preferred_element_type=jnp.float32)
        acc = acc_ref[...]
        z_ref[0, od] = acc.reshape(Ho, Wo, Cout)
        zg = acc.reshape(-1, 8, Cout)
        s_ref[...] += jnp.sum(zg, axis=0)
        sq_ref[...] += jnp.sum(zg * zg, axis=0)


def _conv_s3(xp, wm, b, *, Do, Ho, Wo, Cin, Cout):
    N = xp.shape[0]
    _, _, Hp, Wp, _ = xp.shape
    Dh = Do // 2
    half = (N * Dh) // 2

    def xmapj(j):
        def m(c, i):
            g = c * half + i
            return (g // Dh, 2 * (g % Dh) + j, 0, 0, 0)
        return m

    def zmap(c, i):
        g = c * half + i
        return (g // Dh, g % Dh, 0, 0, 0)

    kern = functools.partial(_conv_s3_kernel, Ho=Ho, Wo=Wo, Cin=Cin, Cout=Cout)
    return pl.pallas_call(
        kern,
        out_shape=(jax.ShapeDtypeStruct((N, Do, Ho, Wo, Cout), jnp.float32),
                   jax.ShapeDtypeStruct((16, Cout), jnp.float32),
                   jax.ShapeDtypeStruct((16, Cout), jnp.float32)),
        grid=(2, half),
        in_specs=[pl.BlockSpec((1, 1, Hp, Wp, Cin), xmapj(0)),
                  pl.BlockSpec((1, 1, Hp, Wp, Cin), xmapj(1)),
                  pl.BlockSpec((1, 1, Hp, Wp, Cin), xmapj(2)),
                  pl.BlockSpec((1, 1, Hp, Wp, Cin), xmapj(3)),
                  pl.BlockSpec((3, 9 * Cin, Cout), lambda c, i: (0, 0, 0)),
                  pl.BlockSpec((1, Cout), lambda c, i: (0, 0))],
        out_specs=(pl.BlockSpec((1, 2, Ho, Wo, Cout), zmap),
                   pl.BlockSpec((8, Cout), lambda c, i: (c, 0)),
                   pl.BlockSpec((8, Cout), lambda c, i: (c, 0))),
        scratch_shapes=[pltpu.VMEM((Ho * Wo, 9 * Cin), jnp.bfloat16),
                        pltpu.VMEM((Ho * Wo, Cout), jnp.float32)],
        compiler_params=_cp("parallel", "arbitrary"),
    )(xp, xp, xp, xp, wm, b)


def _s2_post_kernel(z_ref, sc_ref, sh_ref, o_ref, *, slope, dpad):
    g = pl.program_id(0) * pl.num_programs(1) + pl.program_id(1)
    dp = g % dpad
    interior = (dp >= 1) & (dp <= dpad - 2)

    @pl.when(interior)
    def _():
        y = z_ref[0, 0].astype(jnp.float32) * sc_ref[0] + sh_ref[0]
        y = jnp.where(y >= 0, y, slope * y).astype(jnp.bfloat16)
        H, W, C = y.shape
        zr = jnp.zeros((1, W, C), jnp.bfloat16)
        y = jnp.concatenate([zr, y, zr], axis=0)
        zc = jnp.zeros((H + 2, 1, C), jnp.bfloat16)
        o_ref[0, 0] = jnp.concatenate([zc, y, zc], axis=1)

    @pl.when(jnp.logical_not(interior))
    def _():
        o_ref[...] = jnp.zeros_like(o_ref)


def _s2_post(z5, sc, sh, slope):
    N, D, H, W, C = z5.shape
    dpad = D + 2
    half = (N * dpad) // 2

    def zmap(c, i):
        g = c * half + i
        return (g // dpad, jnp.clip(g % dpad - 1, 0, D - 1), 0, 0, 0)

    def omap(c, i):
        g = c * half + i
        return (g // dpad, g % dpad, 0, 0, 0)

    return pl.pallas_call(
        functools.partial(_s2_post_kernel, slope=slope, dpad=dpad),
        out_shape=jax.ShapeDtypeStruct((N, dpad, H + 2, W + 2, C), jnp.bfloat16),
        grid=(2, half),
        in_specs=[pl.BlockSpec((1, 1, H, W, C), zmap),
                  pl.BlockSpec((1, C), lambda c, i: (0, 0)),
                  pl.BlockSpec((1, C), lambda c, i: (0, 0))],
        out_specs=pl.BlockSpec((1, 1, H + 2, W + 2, C), omap),
        compiler_params=_cp("parallel", "arbitrary"),
    )(z5, sc, sh)


def _tail_kernel(z_ref, sc_ref, sh_ref, w_ref, o_ref, *, slope):
    @pl.when(pl.program_id(1) == 0)
    def _():
        o_ref[...] = jnp.zeros_like(o_ref)

    def act(v):
        u = v.astype(jnp.float32) * sc_ref[0] + sh_ref[0]
        return jnp.where(u >= 0, u, slope * u)

    y = act(z_ref[0, 0]) + act(z_ref[0, 1])
    H, W, C = y.shape
    y = y.reshape(H // 2, 2, W, C)
    y = (y[:, 0] + y[:, 1]) * 0.25
    s = jnp.sum(y * w_ref[0])
    o_ref[...] += jnp.full(o_ref.shape, s, jnp.float32)


def _tail(z3, sc, sh, wv, slope):
    N, D, H, W, C = z3.shape
    D2 = D // 2
    out = pl.pallas_call(
        functools.partial(_tail_kernel, slope=slope),
        out_shape=jax.ShapeDtypeStruct((N, 8, 128), jnp.float32),
        grid=(N, D2),
        in_specs=[pl.BlockSpec((1, 2, H, W, C), lambda n, d: (n, d, 0, 0, 0)),
                  pl.BlockSpec((1, C), lambda n, d: (0, 0)),
                  pl.BlockSpec((1, C), lambda n, d: (0, 0)),
                  pl.BlockSpec((1, H // 2, W, C), lambda n, d: (d, 0, 0, 0))],
        out_specs=pl.BlockSpec((1, 8, 128), lambda n, d: (n, 0, 0)),
        compiler_params=_cp("parallel", "arbitrary"),
    )(z3, sc, sh, wv)
    return out[:, 0, :1]


def _im2colT_c1_bf16(x, stride):
    N, D, H, W = x.shape
    Do = (D - 1) // stride + 1
    Ho = (H - 1) // stride + 1
    Wo = (W - 1) // stride + 1
    xp = jnp.pad(x, ((0, 0), (1, 1), (1, 1), (1, 1)))
    M = N * Do * Ho * Wo
    rows = []
    for kd in range(3):
        for kh in range(3):
            for kw in range(3):
                sl = xp[:, kd:kd + (Do - 1) * stride + 1:stride,
                        kh:kh + (Ho - 1) * stride + 1:stride,
                        kw:kw + (Wo - 1) * stride + 1:stride]
                rows.append(sl.astype(jnp.bfloat16).reshape(1, M))
    return jnp.concatenate(rows, axis=0), (Do, Ho, Wo)


def _affine(s, sq, count, gamma, beta, eps):
    s = jnp.sum(s, axis=0)
    sq = jnp.sum(sq, axis=0)
    mean = s / count
    var = sq / count - mean * mean
    scale = gamma / jnp.sqrt(var + eps)
    shift = beta - mean * scale
    C = scale.shape[0]
    return scale.reshape(1, C), shift.reshape(1, C)


def kernel(x, w1, b1, g1, be1, w2, b2, g2, be2, w3, b3, g3, be3, wl, bl):
    eps, slope = 1e-5, 0.2
    N = x.shape[0]
    bf = jnp.bfloat16

    pT, (Do1, Ho1, Wo1) = _im2colT_c1_bf16(x[:, 0], stride=2)
    M1 = pT.shape[1]
    G, rs = _s1_gram(pT, tm=16384)
    w1f = w1.reshape(32, 27).T
    wr = w1f.T @ rs
    s1 = wr + M1 * b1
    sq1 = jnp.einsum("kc,kl,lc->c", w1f, G, w1f) + 2.0 * b1 * wr + M1 * b1 * b1
    mean1 = s1 / M1
    var1 = sq1 / M1 - mean1 * mean1
    sc1 = g1 / jnp.sqrt(var1 + eps)
    sh1 = be1 - mean1 * sc1
    xph2 = _s1_post(pT, w1f.astype(bf), b1.reshape(1, 32),
                    sc1.reshape(1, 32), sh1.reshape(1, 32), slope, N, Do1)

    w2m = w2.transpose(2, 3, 4, 1, 0).reshape(3, 9 * 32, 64).astype(bf)
    Do2, Ho2, Wo2 = Do1 // 2, Ho1 // 2, Wo1 // 2
    z2, s2, sq2 = _conv_s2(xph2, w2m, b2.reshape(1, 64),
                           Do=Do2, Ho=Ho2, Wo=Wo2, Cin=32, Cout=64)
    sc2, sh2 = _affine(s2, sq2, N * Do2 * Ho2 * Wo2, g2, be2, eps)
    x3 = _s2_post(z2, sc2, sh2, slope)

    w3m = w3.transpose(2, 3, 4, 1, 0).reshape(3, 9 * 64, 128).astype(bf)
    z3, s3, sq3 = _conv_s3(x3, w3m, b3.reshape(1, 128),
                           Do=Do2, Ho=Ho2, Wo=Wo2, Cin=64, Cout=128)
    sc3, sh3 = _affine(s3, sq3, N * Do2 * Ho2 * Wo2, g3, be3, eps)

    D2, H2, W2 = Do2 // 2, Ho2 // 2, Wo2 // 2
    wv = wl.reshape(128, D2, H2, W2)
    wv = jnp.repeat(wv, 2, axis=3) * 0.5
    wv = wv.transpose(1, 2, 3, 0)
    out = _tail(z3, sc3, sh3, wv, slope)
    return out + bl[None, :]
